# baseline (speedup 1.0000x reference)
import functools

import jax
import jax.numpy as jnp
from jax.experimental import pallas as pl
from jax.experimental.pallas import tpu as pltpu

K_EDGE = 512
KH = 256
BD = 128
S_STRIDE = 521
SENT = 1 << 22
BM_MM = 1024


def _matmul_kernel(x_ref, w_ref, o_ref):
    r = jnp.dot(x_ref[...], w_ref[...], preferred_element_type=jnp.float32)
    m2 = o_ref.shape[0]
    o_ref[0:m2:2, :] = r[:, 0:128]
    o_ref[1:m2:2, :] = r[:, 128:256]


def _agg_kernel(sd_ref, dst_ref, tbl_ref, wo_ref, src_ref, o_ref,
                gt_ref, acca_ref, accb_ref, deg_ref, sbuf_ref, sems, *, fuse_wout):
    b = pl.program_id(0)
    acca_ref[...] = jnp.zeros_like(acca_ref)
    accb_ref[...] = jnp.zeros_like(accb_ref)
    deg_ref[...] = jnp.zeros_like(deg_ref)
    base = b * BD

    e_lo = sd_ref[b]
    e_hi = sd_ref[b + 1]
    c_lo = e_lo // K_EDGE
    c_hi = (e_hi + K_EDGE - 1) // K_EDGE

    @pl.when(c_lo < c_hi)
    def _():
        pltpu.make_async_copy(src_ref.at[c_lo], sbuf_ref.at[0], sems.at[0]).start()

        def chunk_body(c, slot):
            pltpu.make_async_copy(src_ref.at[c], sbuf_ref.at[slot], sems.at[slot]).wait()
            nxt = 1 - slot

            @pl.when(c + 1 < c_hi)
            def _():
                pltpu.make_async_copy(src_ref.at[c + 1], sbuf_ref.at[nxt], sems.at[nxt]).start()

            for mi in range(K_EDGE):
                i = pl.multiple_of(sbuf_ref[slot, 0, mi], 2)
                gt_ref[mi:mi + 2 * S_STRIDE:S_STRIDE, :] = tbl_ref[pl.ds(i, 2), :]

            dloc = dst_ref[c] - base
            iota = jax.lax.broadcasted_iota(jnp.int32, (BD, KH), 0)
            oh_a = jnp.where(dloc[:, 0:KH] == iota, 1.0, 0.0)
            oh_b = jnp.where(dloc[:, KH:K_EDGE] == iota, 1.0, 0.0)
            deg_ref[:, 0:KH] += oh_a
            deg_ref[:, KH:K_EDGE] += oh_b
            acca_ref[:, 0:128] += jnp.dot(oh_a, gt_ref[pl.ds(0, KH), :],
                                          preferred_element_type=jnp.float32)
            acca_ref[:, 128:256] += jnp.dot(oh_a, gt_ref[pl.ds(S_STRIDE, KH), :],
                                            preferred_element_type=jnp.float32)
            accb_ref[:, 0:128] += jnp.dot(oh_b, gt_ref[pl.ds(KH, KH), :],
                                          preferred_element_type=jnp.float32)
            accb_ref[:, 128:256] += jnp.dot(oh_b, gt_ref[pl.ds(S_STRIDE + KH, KH), :],
                                            preferred_element_type=jnp.float32)
            return nxt

        jax.lax.fori_loop(c_lo, c_hi, chunk_body, 0)

    deg = jnp.sum(deg_ref[...], axis=1, keepdims=True)
    r = (acca_ref[...] + accb_ref[...]) / deg
    if fuse_wout:
        o_ref[...] = jnp.dot(r, wo_ref[...], preferred_element_type=jnp.float32)
    else:
        m2 = o_ref.shape[0]
        o_ref[0:m2:2, :] = r[:, 0:128]
        o_ref[1:m2:2, :] = r[:, 128:256]


def kernel(x, edges, node_graph_ind, W_hidden, W_out):
    n = x.shape[0]
    f_in = x.shape[1]
    f_hid = W_hidden.shape[1]
    f_out = W_out.shape[1]
    npad = ((n + BM_MM - 1) // BM_MM) * BM_MM
    nb = npad // BD

    a, b2 = edges[:, 0], edges[:, 1]
    u = jnp.minimum(a, b2)
    v = jnp.maximum(a, b2)
    keys = jnp.concatenate([u * n + v, v * n + u])
    skey = jnp.sort(keys)
    prev = jnp.concatenate([jnp.full((1,), -1, jnp.int32), skey[:-1]])
    dup = skey == prev
    dst = skey // n
    src = skey - dst * n
    dstc = jnp.where(dup, SENT, dst)
    e_tot = skey.shape[0]
    nchunk = (e_tot + K_EDGE - 1) // K_EDGE
    pad = nchunk * K_EDGE - e_tot
    dst_arr = jnp.pad(dstc, (0, pad), constant_values=1 << 22).reshape(nchunk, 1, K_EDGE)
    src_arr = (jnp.pad(src, (0, pad)) * 2).reshape(nchunk, 1, K_EDGE)
    bounds = (jnp.arange(nb + 1, dtype=jnp.int32) * BD) * n
    sd = jnp.searchsorted(skey, bounds).astype(jnp.int32)

    x_pad = jnp.pad(x, ((0, npad - n), (0, 0)))

    xwi = pl.pallas_call(
        _matmul_kernel,
        grid=(npad // BM_MM,),
        in_specs=[
            pl.BlockSpec((BM_MM, f_in), lambda i: (i, 0)),
            pl.BlockSpec((f_in, f_hid), lambda i: (0, 0)),
        ],
        out_specs=pl.BlockSpec((2 * BM_MM, 128), lambda i: (i, 0)),
        out_shape=jax.ShapeDtypeStruct((2 * npad, 128), jnp.float32),
        compiler_params=pltpu.CompilerParams(
            dimension_semantics=("parallel",),
            vmem_limit_bytes=40 * 1024 * 1024,
        ),
    )(x_pad, W_hidden)

    def agg_call(table, wo, fuse_wout):
        if fuse_wout:
            out_spec = pl.BlockSpec((BD, f_out), lambda i: (i, 0))
            out_shape = jax.ShapeDtypeStruct((npad, f_out), jnp.float32)
        else:
            out_spec = pl.BlockSpec((2 * BD, 128), lambda i: (i, 0))
            out_shape = jax.ShapeDtypeStruct((2 * npad, 128), jnp.float32)
        return pl.pallas_call(
            functools.partial(_agg_kernel, fuse_wout=fuse_wout),
            grid=(nb,),
            in_specs=[
                pl.BlockSpec(memory_space=pltpu.SMEM),
                pl.BlockSpec((nchunk, 1, K_EDGE), lambda i: (0, 0, 0)),
                pl.BlockSpec((2 * npad, 128), lambda i: (0, 0)),
                pl.BlockSpec((f_hid, f_out), lambda i: (0, 0)),
                pl.BlockSpec(memory_space=pl.ANY),
            ],
            out_specs=out_spec,
            out_shape=out_shape,
            scratch_shapes=[
                pltpu.VMEM((S_STRIDE + K_EDGE, 128), jnp.float32),
                pltpu.VMEM((BD, f_hid), jnp.float32),
                pltpu.VMEM((BD, f_hid), jnp.float32),
                pltpu.VMEM((BD, K_EDGE), jnp.float32),
                pltpu.SMEM((2, 1, K_EDGE), jnp.int32),
                pltpu.SemaphoreType.DMA((2,)),
            ],
            compiler_params=pltpu.CompilerParams(
                dimension_semantics=("parallel",),
                vmem_limit_bytes=60 * 1024 * 1024,
                disable_bounds_checks=True,
            ),
        )(sd, dst_arr, table, wo, src_arr)

    h2i = agg_call(xwi, W_out, fuse_wout=False)
    out = agg_call(h2i, W_out, fuse_wout=True)
    return out[:n]


# speedup vs baseline: 1.0025x; 1.0025x over previous
import functools

import jax
import jax.numpy as jnp
from jax.experimental import pallas as pl
from jax.experimental.pallas import tpu as pltpu

K_EDGE = 1024
KH = 512
BD = 128
S_STRIDE = 1033
SENT = 1 << 22
BM_MM = 1024


def _matmul_kernel(x_ref, w_ref, o_ref):
    r = jnp.dot(x_ref[...], w_ref[...], preferred_element_type=jnp.float32)
    m2 = o_ref.shape[0]
    o_ref[0:m2:2, :] = r[:, 0:128]
    o_ref[1:m2:2, :] = r[:, 128:256]


def _agg_kernel(sd_ref, dst_ref, tbl_ref, wo_ref, src_ref, o_ref,
                gt_ref, acca_ref, accb_ref, deg_ref, sbuf_ref, sems, *, fuse_wout):
    b = pl.program_id(0)
    acca_ref[...] = jnp.zeros_like(acca_ref)
    accb_ref[...] = jnp.zeros_like(accb_ref)
    deg_ref[...] = jnp.zeros_like(deg_ref)
    base = b * BD

    e_lo = sd_ref[b]
    e_hi = sd_ref[b + 1]
    c_lo = e_lo // K_EDGE
    c_hi = (e_hi + K_EDGE - 1) // K_EDGE

    @pl.when(c_lo < c_hi)
    def _():
        pltpu.make_async_copy(src_ref.at[c_lo], sbuf_ref.at[0], sems.at[0]).start()

        def chunk_body(c, slot):
            pltpu.make_async_copy(src_ref.at[c], sbuf_ref.at[slot], sems.at[slot]).wait()
            nxt = 1 - slot

            @pl.when(c + 1 < c_hi)
            def _():
                pltpu.make_async_copy(src_ref.at[c + 1], sbuf_ref.at[nxt], sems.at[nxt]).start()

            for mi in range(K_EDGE):
                i = pl.multiple_of(sbuf_ref[slot, 0, mi], 2)
                gt_ref[mi:mi + 2 * S_STRIDE:S_STRIDE, :] = tbl_ref[pl.ds(i, 2), :]

            dloc = dst_ref[c] - base
            iota = jax.lax.broadcasted_iota(jnp.int32, (BD, KH), 0)
            oh_a = jnp.where(dloc[:, 0:KH] == iota, 1.0, 0.0)
            oh_b = jnp.where(dloc[:, KH:K_EDGE] == iota, 1.0, 0.0)
            deg_ref[:, 0:KH] += oh_a
            deg_ref[:, KH:K_EDGE] += oh_b
            acca_ref[:, 0:128] += jnp.dot(oh_a, gt_ref[pl.ds(0, KH), :],
                                          preferred_element_type=jnp.float32)
            acca_ref[:, 128:256] += jnp.dot(oh_a, gt_ref[pl.ds(S_STRIDE, KH), :],
                                            preferred_element_type=jnp.float32)
            accb_ref[:, 0:128] += jnp.dot(oh_b, gt_ref[pl.ds(KH, KH), :],
                                          preferred_element_type=jnp.float32)
            accb_ref[:, 128:256] += jnp.dot(oh_b, gt_ref[pl.ds(S_STRIDE + KH, KH), :],
                                            preferred_element_type=jnp.float32)
            return nxt

        jax.lax.fori_loop(c_lo, c_hi, chunk_body, 0)

    deg = jnp.sum(deg_ref[...], axis=1, keepdims=True)
    r = (acca_ref[...] + accb_ref[...]) / deg
    if fuse_wout:
        o_ref[...] = jnp.dot(r, wo_ref[...], preferred_element_type=jnp.float32)
    else:
        m2 = o_ref.shape[0]
        o_ref[0:m2:2, :] = r[:, 0:128]
        o_ref[1:m2:2, :] = r[:, 128:256]


def kernel(x, edges, node_graph_ind, W_hidden, W_out):
    n = x.shape[0]
    f_in = x.shape[1]
    f_hid = W_hidden.shape[1]
    f_out = W_out.shape[1]
    npad = ((n + BM_MM - 1) // BM_MM) * BM_MM
    nb = npad // BD

    a, b2 = edges[:, 0], edges[:, 1]
    u = jnp.minimum(a, b2)
    v = jnp.maximum(a, b2)
    keys = jnp.concatenate([u * n + v, v * n + u])
    skey = jnp.sort(keys)
    prev = jnp.concatenate([jnp.full((1,), -1, jnp.int32), skey[:-1]])
    dup = skey == prev
    dst = skey // n
    src = skey - dst * n
    dstc = jnp.where(dup, SENT, dst)
    e_tot = skey.shape[0]
    nchunk = (e_tot + K_EDGE - 1) // K_EDGE
    pad = nchunk * K_EDGE - e_tot
    dst_arr = jnp.pad(dstc, (0, pad), constant_values=1 << 22).reshape(nchunk, 1, K_EDGE)
    src_arr = (jnp.pad(src, (0, pad)) * 2).reshape(nchunk, 1, K_EDGE)
    bounds = (jnp.arange(nb + 1, dtype=jnp.int32) * BD) * n
    sd = jnp.searchsorted(skey, bounds).astype(jnp.int32)

    x_pad = jnp.pad(x, ((0, npad - n), (0, 0)))

    xwi = pl.pallas_call(
        _matmul_kernel,
        grid=(npad // BM_MM,),
        in_specs=[
            pl.BlockSpec((BM_MM, f_in), lambda i: (i, 0)),
            pl.BlockSpec((f_in, f_hid), lambda i: (0, 0)),
        ],
        out_specs=pl.BlockSpec((2 * BM_MM, 128), lambda i: (i, 0)),
        out_shape=jax.ShapeDtypeStruct((2 * npad, 128), jnp.float32),
        compiler_params=pltpu.CompilerParams(
            dimension_semantics=("parallel",),
            vmem_limit_bytes=40 * 1024 * 1024,
        ),
    )(x_pad, W_hidden)

    def agg_call(table, wo, fuse_wout):
        if fuse_wout:
            out_spec = pl.BlockSpec((BD, f_out), lambda i: (i, 0))
            out_shape = jax.ShapeDtypeStruct((npad, f_out), jnp.float32)
        else:
            out_spec = pl.BlockSpec((2 * BD, 128), lambda i: (i, 0))
            out_shape = jax.ShapeDtypeStruct((2 * npad, 128), jnp.float32)
        return pl.pallas_call(
            functools.partial(_agg_kernel, fuse_wout=fuse_wout),
            grid=(nb,),
            in_specs=[
                pl.BlockSpec(memory_space=pltpu.SMEM),
                pl.BlockSpec((nchunk, 1, K_EDGE), lambda i: (0, 0, 0)),
                pl.BlockSpec((2 * npad, 128), lambda i: (0, 0)),
                pl.BlockSpec((f_hid, f_out), lambda i: (0, 0)),
                pl.BlockSpec(memory_space=pl.ANY),
            ],
            out_specs=out_spec,
            out_shape=out_shape,
            scratch_shapes=[
                pltpu.VMEM((S_STRIDE + K_EDGE, 128), jnp.float32),
                pltpu.VMEM((BD, f_hid), jnp.float32),
                pltpu.VMEM((BD, f_hid), jnp.float32),
                pltpu.VMEM((BD, K_EDGE), jnp.float32),
                pltpu.SMEM((2, 1, K_EDGE), jnp.int32),
                pltpu.SemaphoreType.DMA((2,)),
            ],
            compiler_params=pltpu.CompilerParams(
                dimension_semantics=("parallel",),
                vmem_limit_bytes=60 * 1024 * 1024,
                disable_bounds_checks=True,
            ),
        )(sd, dst_arr, table, wo, src_arr)

    h2i = agg_call(xwi, W_out, fuse_wout=False)
    out = agg_call(h2i, W_out, fuse_wout=True)
    return out[:n]
